# baseline (speedup 1.0000x reference)
import functools

import jax
import jax.numpy as jnp
import numpy as np
from jax import lax
from jax.experimental import pallas as pl
from jax.experimental.pallas import tpu as pltpu
from jax.experimental.pallas import tpu_sc as plsc

D_MODEL = 1024
MAX_LEN = 2048
BATCH = 4

NUM_CORES = 2
NUM_SUBCORES = 16
NUM_WORKERS = NUM_CORES * NUM_SUBCORES

TOTAL_ROWS = BATCH * MAX_LEN
ROWS_PER_WORKER = TOTAL_ROWS // NUM_WORKERS
CHUNK = 32
CHUNKS_PER_WORKER = ROWS_PER_WORKER // CHUNK
LANES = 16
GROUPS = D_MODEL // LANES


def _pos_encoding(max_len, d_model):
    pos = np.arange(max_len)[:, np.newaxis]
    depth = np.arange(d_model / 2)[np.newaxis, :] / (d_model / 2)
    angle_rates = 1.0 / 10000 ** depth
    inner = pos * angle_rates
    pe = np.stack((np.sin(inner), np.cos(inner)), axis=2).reshape((max_len, -1))
    return np.asarray(pe, dtype=np.float32)


_POS_ENC = _pos_encoding(MAX_LEN, D_MODEL)


@functools.partial(
    pl.kernel,
    mesh=plsc.VectorSubcoreMesh(core_axis_name="c", subcore_axis_name="s"),
    out_type=jax.ShapeDtypeStruct((TOTAL_ROWS, D_MODEL), jnp.float32),
    scratch_types=[
        pltpu.VMEM((CHUNK,), jnp.int32),
        pltpu.VMEM((CHUNK, D_MODEL), jnp.float32),
        pltpu.VMEM((CHUNK, D_MODEL), jnp.float32),
        pltpu.SemaphoreType.DMA,
    ],
)
def _sc_embed(idx_hbm, pe_hbm, table_hbm, out_hbm, idx_v, pe_v, rows_v, sem):
    wid = lax.axis_index("s") * NUM_CORES + lax.axis_index("c")
    chunk_rows_per_len = MAX_LEN // CHUNK

    def step(c, _):
        r = wid * CHUNKS_PER_WORKER + c
        flat_base = r * CHUNK
        pos_base = lax.rem(r, chunk_rows_per_len) * CHUNK
        pltpu.sync_copy(idx_hbm.at[r], idx_v)
        pltpu.sync_copy(pe_hbm.at[pl.ds(pos_base, CHUNK)], pe_v)
        pltpu.async_copy(table_hbm.at[idx_v], rows_v, sem).wait()

        def add_row(j, _):
            for k in range(GROUPS):
                sl = pl.ds(k * LANES, LANES)
                rows_v[j, sl] = rows_v[j, sl] + pe_v[j, sl]
            return ()

        lax.fori_loop(0, CHUNK, add_row, ())
        pltpu.sync_copy(rows_v, out_hbm.at[pl.ds(flat_base, CHUNK)])
        return ()

    lax.fori_loop(0, CHUNKS_PER_WORKER, step, ())


def kernel(inputs, table):
    idx2d = inputs.reshape(TOTAL_ROWS // CHUNK, CHUNK)
    out = _sc_embed(idx2d, _POS_ENC, table)
    return out.reshape(BATCH, MAX_LEN, D_MODEL)


# speedup vs baseline: 1.4181x; 1.4181x over previous
import functools

import jax
import jax.numpy as jnp
import numpy as np
from jax import lax
from jax.experimental import pallas as pl
from jax.experimental.pallas import tpu as pltpu
from jax.experimental.pallas import tpu_sc as plsc

D_MODEL = 1024
MAX_LEN = 2048
BATCH = 4

NUM_CORES = 2
NUM_SUBCORES = 16
NUM_WORKERS = NUM_CORES * NUM_SUBCORES

TOTAL_ROWS = BATCH * MAX_LEN
CHUNK = 32
POS_BLOCKS = MAX_LEN // CHUNK
BLOCKS_PER_WORKER = POS_BLOCKS // NUM_WORKERS
STEPS = BLOCKS_PER_WORKER * BATCH
LANES = 16
GROUPS = D_MODEL // LANES


def _pos_encoding(max_len, d_model):
    pos = np.arange(max_len)[:, np.newaxis]
    depth = np.arange(d_model / 2)[np.newaxis, :] / (d_model / 2)
    angle_rates = 1.0 / 10000 ** depth
    inner = pos * angle_rates
    pe = np.stack((np.sin(inner), np.cos(inner)), axis=2).reshape((max_len, -1))
    return np.asarray(pe, dtype=np.float32)


_POS_ENC = _pos_encoding(MAX_LEN, D_MODEL)


@functools.partial(
    pl.kernel,
    mesh=plsc.VectorSubcoreMesh(core_axis_name="c", subcore_axis_name="s"),
    out_type=jax.ShapeDtypeStruct((TOTAL_ROWS, D_MODEL), jnp.float32),
    scratch_types=[
        pltpu.VMEM((STEPS, CHUNK), jnp.int32),
        pltpu.VMEM((CHUNK, D_MODEL), jnp.float32),
        pltpu.VMEM((CHUNK, D_MODEL), jnp.float32),
        pltpu.VMEM((CHUNK, D_MODEL), jnp.float32),
        pltpu.SemaphoreType.DMA,
        pltpu.SemaphoreType.DMA,
        pltpu.SemaphoreType.DMA,
        pltpu.SemaphoreType.DMA,
        pltpu.SemaphoreType.DMA,
        pltpu.SemaphoreType.DMA,
    ],
)
def _sc_embed(idx_hbm, pe_hbm, table_hbm, out_hbm,
              idx_v, pe_v, rows0, rows1,
              sem_i, sem_pe, sem_g0, sem_g1, sem_o0, sem_o1):
    wid = lax.axis_index("s") * NUM_CORES + lax.axis_index("c")
    rows_bufs = (rows0, rows1)
    g_sems = (sem_g0, sem_g1)
    o_sems = (sem_o0, sem_o1)

    def pos_block(s):
        return wid * BLOCKS_PER_WORKER + s // BATCH

    def idx_row(s):
        return (s % BATCH) * POS_BLOCKS + pos_block(s)

    def out_base(s):
        return (s % BATCH) * MAX_LEN + pos_block(s) * CHUNK

    idx_handles = [
        pltpu.async_copy(idx_hbm.at[idx_row(s)], idx_v.at[s], sem_i)
        for s in range(STEPS)
    ]
    for h in idx_handles:
        h.wait()

    pe_h = pltpu.async_copy(
        pe_hbm.at[pl.ds(pos_block(0) * CHUNK, CHUNK)], pe_v, sem_pe)
    gather_h = [None] * STEPS
    gather_h[0] = pltpu.async_copy(table_hbm.at[idx_v.at[0]], rows0, sem_g0)

    out_h = [None] * STEPS
    for s in range(STEPS):
        buf = s % 2
        if s + 1 < STEPS:
            if s >= 1:
                out_h[s - 1].wait()
            gather_h[s + 1] = pltpu.async_copy(
                table_hbm.at[idx_v.at[s + 1]],
                rows_bufs[(s + 1) % 2], g_sems[(s + 1) % 2])
        if s == 0 or s == BATCH:
            pe_h.wait()
        gather_h[s].wait()

        rv = rows_bufs[buf]

        def add_row(j, _):
            for k in range(GROUPS):
                sl = pl.ds(k * LANES, LANES)
                rv[j, sl] = rv[j, sl] + pe_v[j, sl]
            return ()

        lax.fori_loop(0, CHUNK, add_row, ())

        out_h[s] = pltpu.async_copy(
            rv, out_hbm.at[pl.ds(out_base(s), CHUNK)], o_sems[buf])

        if s == BATCH - 1:
            pe_h = pltpu.async_copy(
                pe_hbm.at[pl.ds(pos_block(BATCH) * CHUNK, CHUNK)],
                pe_v, sem_pe)

    out_h[STEPS - 2].wait()
    out_h[STEPS - 1].wait()


def kernel(inputs, table):
    idx2d = inputs.reshape(TOTAL_ROWS // CHUNK, CHUNK)
    out = _sc_embed(idx2d, _POS_ENC, table)
    return out.reshape(BATCH, MAX_LEN, D_MODEL)


# speedup vs baseline: 1.6167x; 1.1400x over previous
import functools

import jax
import jax.numpy as jnp
import numpy as np
from jax import lax
from jax.experimental import pallas as pl
from jax.experimental.pallas import tpu as pltpu
from jax.experimental.pallas import tpu_sc as plsc

D_MODEL = 1024
MAX_LEN = 2048
BATCH = 4

NUM_CORES = 2
NUM_SUBCORES = 16
NUM_WORKERS = NUM_CORES * NUM_SUBCORES

CHUNK = 32
POS_BLOCKS = MAX_LEN // CHUNK
BLOCKS_PER_WORKER = POS_BLOCKS // NUM_WORKERS
STEPS = BLOCKS_PER_WORKER * BATCH
LANES = 16
GROUPS = D_MODEL // LANES


def _pos_encoding(max_len, d_model):
    pos = np.arange(max_len)[:, np.newaxis]
    depth = np.arange(d_model / 2)[np.newaxis, :] / (d_model / 2)
    angle_rates = 1.0 / 10000 ** depth
    inner = pos * angle_rates
    pe = np.stack((np.sin(inner), np.cos(inner)), axis=2).reshape((max_len, -1))
    return np.asarray(pe, dtype=np.float32)


_POS_ENC_NP = _pos_encoding(MAX_LEN, D_MODEL)
_POS_ENC_DEV = None


@functools.partial(
    pl.kernel,
    mesh=plsc.VectorSubcoreMesh(core_axis_name="c", subcore_axis_name="s"),
    out_type=jax.ShapeDtypeStruct((BATCH, MAX_LEN, D_MODEL), jnp.float32),
    scratch_types=[
        pltpu.VMEM((STEPS, CHUNK), jnp.int32),
        pltpu.VMEM((CHUNK, D_MODEL), jnp.float32),
        pltpu.VMEM((CHUNK, D_MODEL), jnp.float32),
        pltpu.VMEM((CHUNK, D_MODEL), jnp.float32),
        pltpu.SemaphoreType.DMA,
        pltpu.SemaphoreType.DMA,
        pltpu.SemaphoreType.DMA,
        pltpu.SemaphoreType.DMA,
        pltpu.SemaphoreType.DMA,
        pltpu.SemaphoreType.DMA,
    ],
)
def _sc_embed(idx_hbm, pe_hbm, table_hbm, out_hbm,
              idx_v, pe_v, rows0, rows1,
              sem_i, sem_pe, sem_g0, sem_g1, sem_o0, sem_o1):
    wid = lax.axis_index("s") * NUM_CORES + lax.axis_index("c")
    rows_bufs = (rows0, rows1)
    g_sems = (sem_g0, sem_g1)
    o_sems = (sem_o0, sem_o1)

    def pos_block(s):
        return wid * BLOCKS_PER_WORKER + s // BATCH

    def batch_of(s):
        return s % BATCH

    idx_h = [None] * STEPS
    idx_h[0] = pltpu.async_copy(
        idx_hbm.at[batch_of(0), pl.ds(pos_block(0) * CHUNK, CHUNK)],
        idx_v.at[0], sem_i)
    idx_h[0].wait()
    gather_h = [None] * STEPS
    gather_h[0] = pltpu.async_copy(table_hbm.at[idx_v.at[0]], rows0, sem_g0)

    for s in range(1, STEPS):
        idx_h[s] = pltpu.async_copy(
            idx_hbm.at[batch_of(s), pl.ds(pos_block(s) * CHUNK, CHUNK)],
            idx_v.at[s], sem_i)
    pe_h = pltpu.async_copy(
        pe_hbm.at[pl.ds(pos_block(0) * CHUNK, CHUNK)], pe_v, sem_pe)

    out_h = [None] * STEPS
    for s in range(STEPS):
        buf = s % 2
        if s + 1 < STEPS:
            if s >= 1:
                out_h[s - 1].wait()
            idx_h[s + 1].wait()
            gather_h[s + 1] = pltpu.async_copy(
                table_hbm.at[idx_v.at[s + 1]],
                rows_bufs[(s + 1) % 2], g_sems[(s + 1) % 2])
        if s == 0 or s == BATCH:
            pe_h.wait()
        gather_h[s].wait()

        rv = rows_bufs[buf]

        @plsc.parallel_loop(0, CHUNK, 1, unroll=1)
        def _(j):
            for k in range(GROUPS):
                sl = pl.ds(k * LANES, LANES)
                rv[j, sl] = rv[j, sl] + pe_v[j, sl]

        out_h[s] = pltpu.async_copy(
            rv, out_hbm.at[batch_of(s), pl.ds(pos_block(s) * CHUNK, CHUNK)],
            o_sems[buf])

        if s == BATCH - 1:
            pe_h = pltpu.async_copy(
                pe_hbm.at[pl.ds(pos_block(BATCH) * CHUNK, CHUNK)],
                pe_v, sem_pe)

    out_h[STEPS - 2].wait()
    out_h[STEPS - 1].wait()


def kernel(inputs, table):
    global _POS_ENC_DEV
    if _POS_ENC_DEV is None:
        _POS_ENC_DEV = jnp.asarray(_POS_ENC_NP)
    return _sc_embed(inputs, _POS_ENC_DEV, table)
